# baseline (speedup 1.0000x reference)
import functools

import jax
import jax.numpy as jnp
from jax import lax
from jax.experimental import pallas as pl
from jax.experimental.pallas import tpu as pltpu
from jax.experimental.pallas import tpu_sc as plsc

_NC = 2
_NS = 16
_NW = _NC * _NS
_L = 16

_GB = 128
_CH = 2048
_BLK = 1024
_BATCH = 16


def _relayout_body(blocks_per_w, tab_hbm, lin_hbm, in_a, in_b, out_a, out_b,
                   sem_ia, sem_ib, sem_oa, sem_ob):
    wid = lax.axis_index("s") * _NC + lax.axis_index("c")
    base_w = wid * blocks_per_w * _BLK
    n_batches = blocks_per_w // _BATCH
    bwords = _BATCH * _BLK
    ins = (in_a, in_b)
    outs = (out_a, out_b)
    sem_i = (sem_ia, sem_ib)
    sem_o = (sem_oa, sem_ob)

    lane = lax.iota(jnp.int32, _L)
    perm = lax.shift_left(lane & 7, 7) + lax.shift_right_logical(lane, 3)

    def fire_in(t):
        off = base_w + t * bwords
        pltpu.async_copy(tab_hbm.at[pl.ds(off, bwords)], ins[t % 2],
                         sem_i[t % 2])

    def transpose_batch(t):
        src = ins[t % 2]
        dst = outs[t % 2]

        def tr_body(g, carry):
            for u in range(4):
                o = g * 64 + u * _L
                blk = lax.shift_left(lax.shift_right_logical(o, 10), 10)
                p0 = lax.shift_right_logical(o & (_BLK - 1), 3)
                idx = perm + (blk + p0)
                dst[pl.ds(o, _L)] = plsc.load_gather(src, [idx])
            return carry

        lax.fori_loop(0, bwords // 64, tr_body, 0)

    fire_in(0)
    for t in range(n_batches):
        if t + 1 < n_batches:
            fire_in(t + 1)
        pltpu.make_async_copy(
            tab_hbm.at[pl.ds(0, bwords)], ins[t % 2], sem_i[t % 2]).wait()
        if t >= 2:
            pltpu.make_async_copy(
                tab_hbm.at[pl.ds(0, bwords)], outs[t % 2], sem_o[t % 2]).wait()
        transpose_batch(t)
        pltpu.async_copy(outs[t % 2],
                         lin_hbm.at[pl.ds(base_w + t * bwords, bwords)],
                         sem_o[t % 2])
    for t in (n_batches - 2, n_batches - 1):
        pltpu.make_async_copy(
            tab_hbm.at[pl.ds(0, bwords)], outs[t % 2], sem_o[t % 2]).wait()


def _fire_gathers(tab2d, idx_v, rows_v, sem, c):
    for j in range(_CH // _GB):
        pltpu.async_copy(
            tab2d.at[idx_v.at[c * (_CH // _GB) + j]],
            rows_v.at[pl.ds(j * _GB, _GB)],
            sem,
        )


def _compute_chunk(rows_v, pbuf, vbuf, zero_f, iota, cols):
    c0, c1, c2, c3, c4, c5 = cols

    def group_body(g, carry):
        row = g * _L + iota
        r0 = plsc.load_gather(rows_v, [row, c0])
        r1 = plsc.load_gather(rows_v, [row, c1])
        r3 = plsc.load_gather(rows_v, [row, c3])
        r4 = plsc.load_gather(rows_v, [row, c4])
        r5 = plsc.load_gather(rows_v, [row, c5])

        ob = lax.shift_left(lax.shift_right_logical(g, 3), 9) \
            + lax.shift_left(g & 7, 4)
        pbuf[pl.ds(ob, _L)] = zero_f
        pbuf[pl.ds(ob + 128, _L)] = 1000.0 * r0
        pbuf[pl.ds(ob + 256, _L)] = 1000.0 * r1

        n2 = r5 * r5 + r3 * r3 + r4 * r4
        i = plsc.bitcast(n2, jnp.int32)
        i = 0x5F3759DF - lax.shift_right_logical(i, 1)
        y = plsc.bitcast(i, jnp.float32)
        xh = 0.5 * n2
        y = y * (1.5 - xh * y * y)
        y = y * (1.5 - xh * y * y)
        y = y * (1.5 - xh * y * y)

        vbuf[pl.ds(ob, _L)] = (zero_f - r5) * y
        vbuf[pl.ds(ob + 128, _L)] = r3 * y
        vbuf[pl.ds(ob + 256, _L)] = r4 * y
        return carry

    lax.fori_loop(0, _CH // _L, group_body, 0)


def _gather_body(b_per_w, n_chunks, tab2d, idx_hbm, p_hbm, v_hbm,
                 idx_v, rows_a, rows_b, pbuf, vbuf, sem_a, sem_b):
    wid = lax.axis_index("s") * _NC + lax.axis_index("c")
    base = wid * b_per_w
    rows_per_w = b_per_w // _GB
    pltpu.sync_copy(idx_hbm.at[pl.ds(wid * rows_per_w, rows_per_w)], idx_v)

    rows = (rows_a, rows_b)
    sems = (sem_a, sem_b)
    zero_f = jnp.zeros((_L,), jnp.float32)
    iota = lax.iota(jnp.int32, _L)
    cols = tuple(jnp.zeros((_L,), jnp.int32) + c for c in range(6))

    _fire_gathers(tab2d, idx_v, rows[0], sems[0], 0)
    for c in range(n_chunks):
        if c + 1 < n_chunks:
            _fire_gathers(tab2d, idx_v, rows[(c + 1) % 2], sems[(c + 1) % 2],
                          c + 1)
        pltpu.make_async_copy(
            tab2d.at[pl.ds(0, _CH)], rows[c % 2], sems[c % 2]).wait()
        _compute_chunk(rows[c % 2], pbuf, vbuf, zero_f, iota, cols)
        out_off = (base + c * _CH) * 4
        pltpu.sync_copy(pbuf, p_hbm.at[pl.ds(out_off, _CH * 4)])
        pltpu.sync_copy(vbuf, v_hbm.at[pl.ds(out_off, _CH * 4)])


def kernel(all_rays, indices):
    n = indices.shape[0]
    m = all_rays.shape[0]
    assert n % (_NW * _CH) == 0
    b_per_w = n // _NW
    n_chunks = b_per_w // _CH
    m2 = ((m + _NW * _BATCH * _GB - 1) // (_NW * _BATCH * _GB)) \
        * (_NW * _BATCH * _GB)
    blocks_per_w = (m2 // _GB) // _NW

    rays_p = jnp.pad(all_rays, ((0, m2 - m), (0, 8 - all_rays.shape[1])))
    tab_flat = (rays_p.reshape(m2 // _GB, _GB, 8)
                .transpose(0, 2, 1)
                .reshape(m2 * 8))
    idx2d = indices.reshape(n // _GB, _GB).astype(jnp.int32)

    mesh = plsc.VectorSubcoreMesh(
        core_axis_name="c", subcore_axis_name="s",
        num_cores=_NC, num_subcores=_NS)
    params = pltpu.CompilerParams(
        needs_layout_passes=False, use_tc_tiling_on_sc=False)

    relayout = pl.kernel(
        functools.partial(_relayout_body, blocks_per_w),
        mesh=mesh,
        out_type=jax.ShapeDtypeStruct((m2 * 8,), jnp.float32),
        scratch_types=[
            pltpu.VMEM((_BATCH * _BLK,), jnp.float32),
            pltpu.VMEM((_BATCH * _BLK,), jnp.float32),
            pltpu.VMEM((_BATCH * _BLK,), jnp.float32),
            pltpu.VMEM((_BATCH * _BLK,), jnp.float32),
            pltpu.SemaphoreType.DMA,
            pltpu.SemaphoreType.DMA,
            pltpu.SemaphoreType.DMA,
            pltpu.SemaphoreType.DMA,
        ],
        compiler_params=params,
    )
    tab_lin = relayout(tab_flat).reshape(m2, 8)

    gather = pl.kernel(
        functools.partial(_gather_body, b_per_w, n_chunks),
        mesh=mesh,
        out_type=(
            jax.ShapeDtypeStruct((n * 4,), jnp.float32),
            jax.ShapeDtypeStruct((n * 4,), jnp.float32),
        ),
        scratch_types=[
            pltpu.VMEM((b_per_w // _GB, _GB), jnp.int32),
            pltpu.VMEM((_CH, 8), jnp.float32),
            pltpu.VMEM((_CH, 8), jnp.float32),
            pltpu.VMEM((_CH * 4,), jnp.float32),
            pltpu.VMEM((_CH * 4,), jnp.float32),
            pltpu.SemaphoreType.DMA,
            pltpu.SemaphoreType.DMA,
        ],
        compiler_params=params,
    )
    p_flat, v_flat = gather(tab_lin, idx2d)

    def unblock(x):
        return (x.reshape(n // _GB, 4, _GB)
                .transpose(0, 2, 1)
                .reshape(n, 4)[:, :3])

    return unblock(p_flat), unblock(v_flat)


# speedup vs baseline: 1.0033x; 1.0033x over previous
import functools

import jax
import jax.numpy as jnp
from jax import lax
from jax.experimental import pallas as pl
from jax.experimental.pallas import tpu as pltpu
from jax.experimental.pallas import tpu_sc as plsc

_NC = 2
_NS = 16
_NW = _NC * _NS
_L = 16

_GB = 128
_CH = 2048
_BLK = 1024
_BATCH = 16


def _relayout_body(blocks_per_w, tab_hbm, lin_hbm, in_a, in_b, out_a, out_b,
                   sem_ia, sem_ib, sem_oa, sem_ob):
    wid = lax.axis_index("s") * _NC + lax.axis_index("c")
    base_w = wid * blocks_per_w * _BLK
    n_batches = blocks_per_w // _BATCH
    bwords = _BATCH * _BLK
    ins = (in_a, in_b)
    outs = (out_a, out_b)
    sem_i = (sem_ia, sem_ib)
    sem_o = (sem_oa, sem_ob)

    lane = lax.iota(jnp.int32, _L)
    perm = lax.shift_left(lane & 7, 7) + lax.shift_right_logical(lane, 3)

    def fire_in(t):
        off = base_w + t * bwords
        pltpu.async_copy(tab_hbm.at[pl.ds(off, bwords)], ins[t % 2],
                         sem_i[t % 2])

    def transpose_batch(t):
        src = ins[t % 2]
        dst = outs[t % 2]

        def tr_body(g, carry):
            for u in range(16):
                o = g * 256 + u * _L
                blk = lax.shift_left(lax.shift_right_logical(o, 10), 10)
                p0 = lax.shift_right_logical(o & (_BLK - 1), 3)
                idx = perm + (blk + p0)
                dst[pl.ds(o, _L)] = plsc.load_gather(src, [idx])
            return carry

        lax.fori_loop(0, bwords // 256, tr_body, 0)

    fire_in(0)
    for t in range(n_batches):
        if t + 1 < n_batches:
            fire_in(t + 1)
        pltpu.make_async_copy(
            tab_hbm.at[pl.ds(0, bwords)], ins[t % 2], sem_i[t % 2]).wait()
        if t >= 2:
            pltpu.make_async_copy(
                tab_hbm.at[pl.ds(0, bwords)], outs[t % 2], sem_o[t % 2]).wait()
        transpose_batch(t)
        pltpu.async_copy(outs[t % 2],
                         lin_hbm.at[pl.ds(base_w + t * bwords, bwords)],
                         sem_o[t % 2])
    for t in (n_batches - 2, n_batches - 1):
        pltpu.make_async_copy(
            tab_hbm.at[pl.ds(0, bwords)], outs[t % 2], sem_o[t % 2]).wait()


def _fire_gathers(tab2d, idx_v, rows_v, sem, c):
    for j in range(_CH // _GB):
        pltpu.async_copy(
            tab2d.at[idx_v.at[c * (_CH // _GB) + j]],
            rows_v.at[pl.ds(j * _GB, _GB)],
            sem,
        )


def _compute_chunk(rows_v, pbuf, vbuf, zero_f, iota, cols):
    c0, c1, c2, c3, c4, c5 = cols

    def group_body(g, carry):
        row = g * _L + iota
        r0 = plsc.load_gather(rows_v, [row, c0])
        r1 = plsc.load_gather(rows_v, [row, c1])
        r3 = plsc.load_gather(rows_v, [row, c3])
        r4 = plsc.load_gather(rows_v, [row, c4])
        r5 = plsc.load_gather(rows_v, [row, c5])

        ob = lax.shift_left(lax.shift_right_logical(g, 3), 9) \
            + lax.shift_left(g & 7, 4)
        pbuf[pl.ds(ob, _L)] = zero_f
        pbuf[pl.ds(ob + 128, _L)] = 1000.0 * r0
        pbuf[pl.ds(ob + 256, _L)] = 1000.0 * r1

        n2 = r5 * r5 + r3 * r3 + r4 * r4
        i = plsc.bitcast(n2, jnp.int32)
        i = 0x5F3759DF - lax.shift_right_logical(i, 1)
        y = plsc.bitcast(i, jnp.float32)
        xh = 0.5 * n2
        y = y * (1.5 - xh * y * y)
        y = y * (1.5 - xh * y * y)
        y = y * (1.5 - xh * y * y)

        vbuf[pl.ds(ob, _L)] = (zero_f - r5) * y
        vbuf[pl.ds(ob + 128, _L)] = r3 * y
        vbuf[pl.ds(ob + 256, _L)] = r4 * y
        return carry

    lax.fori_loop(0, _CH // _L, group_body, 0)


def _gather_body(b_per_w, n_chunks, tab2d, idx_hbm, p_hbm, v_hbm,
                 idx_v, rows_a, rows_b, pbuf, vbuf, sem_a, sem_b):
    wid = lax.axis_index("s") * _NC + lax.axis_index("c")
    base = wid * b_per_w
    rows_per_w = b_per_w // _GB
    pltpu.sync_copy(idx_hbm.at[pl.ds(wid * rows_per_w, rows_per_w)], idx_v)

    rows = (rows_a, rows_b)
    sems = (sem_a, sem_b)
    zero_f = jnp.zeros((_L,), jnp.float32)
    iota = lax.iota(jnp.int32, _L)
    cols = tuple(jnp.zeros((_L,), jnp.int32) + c for c in range(6))

    _fire_gathers(tab2d, idx_v, rows[0], sems[0], 0)
    for c in range(n_chunks):
        if c + 1 < n_chunks:
            _fire_gathers(tab2d, idx_v, rows[(c + 1) % 2], sems[(c + 1) % 2],
                          c + 1)
        pltpu.make_async_copy(
            tab2d.at[pl.ds(0, _CH)], rows[c % 2], sems[c % 2]).wait()
        _compute_chunk(rows[c % 2], pbuf, vbuf, zero_f, iota, cols)
        out_off = (base + c * _CH) * 4
        pltpu.sync_copy(pbuf, p_hbm.at[pl.ds(out_off, _CH * 4)])
        pltpu.sync_copy(vbuf, v_hbm.at[pl.ds(out_off, _CH * 4)])


def kernel(all_rays, indices):
    n = indices.shape[0]
    m = all_rays.shape[0]
    assert n % (_NW * _CH) == 0
    b_per_w = n // _NW
    n_chunks = b_per_w // _CH
    m2 = ((m + _NW * _BATCH * _GB - 1) // (_NW * _BATCH * _GB)) \
        * (_NW * _BATCH * _GB)
    blocks_per_w = (m2 // _GB) // _NW

    rays_p = jnp.pad(all_rays, ((0, m2 - m), (0, 8 - all_rays.shape[1])))
    tab_flat = (rays_p.reshape(m2 // _GB, _GB, 8)
                .transpose(0, 2, 1)
                .reshape(m2 * 8))
    idx2d = indices.reshape(n // _GB, _GB).astype(jnp.int32)

    mesh = plsc.VectorSubcoreMesh(
        core_axis_name="c", subcore_axis_name="s",
        num_cores=_NC, num_subcores=_NS)
    params = pltpu.CompilerParams(
        needs_layout_passes=False, use_tc_tiling_on_sc=False)

    relayout = pl.kernel(
        functools.partial(_relayout_body, blocks_per_w),
        mesh=mesh,
        out_type=jax.ShapeDtypeStruct((m2 * 8,), jnp.float32),
        scratch_types=[
            pltpu.VMEM((_BATCH * _BLK,), jnp.float32),
            pltpu.VMEM((_BATCH * _BLK,), jnp.float32),
            pltpu.VMEM((_BATCH * _BLK,), jnp.float32),
            pltpu.VMEM((_BATCH * _BLK,), jnp.float32),
            pltpu.SemaphoreType.DMA,
            pltpu.SemaphoreType.DMA,
            pltpu.SemaphoreType.DMA,
            pltpu.SemaphoreType.DMA,
        ],
        compiler_params=params,
    )
    tab_lin = relayout(tab_flat).reshape(m2, 8)

    gather = pl.kernel(
        functools.partial(_gather_body, b_per_w, n_chunks),
        mesh=mesh,
        out_type=(
            jax.ShapeDtypeStruct((n * 4,), jnp.float32),
            jax.ShapeDtypeStruct((n * 4,), jnp.float32),
        ),
        scratch_types=[
            pltpu.VMEM((b_per_w // _GB, _GB), jnp.int32),
            pltpu.VMEM((_CH, 8), jnp.float32),
            pltpu.VMEM((_CH, 8), jnp.float32),
            pltpu.VMEM((_CH * 4,), jnp.float32),
            pltpu.VMEM((_CH * 4,), jnp.float32),
            pltpu.SemaphoreType.DMA,
            pltpu.SemaphoreType.DMA,
        ],
        compiler_params=params,
    )
    p_flat, v_flat = gather(tab_lin, idx2d)

    def unblock(x):
        return (x.reshape(n // _GB, 4, _GB)
                .transpose(0, 2, 1)
                .reshape(n, 4)[:, :3])

    return unblock(p_flat), unblock(v_flat)
